# baseline (speedup 1.0000x reference)
import functools

import jax
import jax.numpy as jnp
from jax import lax
from jax.experimental import pallas as pl
from jax.experimental.pallas import tpu as pltpu
from jax.experimental.pallas import tpu_sc as plsc

_NC = 2
_NS = 16
_L = 16
_NW = _NC * _NS

_SCH = 2000
_BK = 80
_NG = _BK // _L
_CAP = _SCH + _BK


def _proj_body(x_ref, w_ref, a_ref, h_ref, t_ref):
    h = jnp.dot(x_ref[...], w_ref[0], preferred_element_type=jnp.float32)
    h_ref[...] = h
    t_ref[...] = jnp.dot(h, a_ref[...], preferred_element_type=jnp.float32)


def _make_sc_kernel(N, E, OUT, R, H):
    HD = OUT // H
    EH = E // _NC
    NCHK = EH // _SCH
    ROWS_PT = N // _NS

    mesh = plsc.VectorSubcoreMesh(core_axis_name="c", subcore_axis_name="s")

    @functools.partial(
        pl.kernel,
        out_type=(
            jax.ShapeDtypeStruct((_NW, ROWS_PT, OUT), jnp.float32),
            jax.ShapeDtypeStruct((_NW, H, _L), jnp.float32),
        ),
        mesh=mesh,
        compiler_params=pltpu.CompilerParams(
            needs_layout_passes=False, use_tc_tiling_on_sc=False),
        scratch_types=[
            pltpu.VMEM((ROWS_PT, OUT), jnp.float32),
            pltpu.VMEM((2, _SCH), jnp.int32),
            pltpu.VMEM((2, _SCH), jnp.int32),
            pltpu.VMEM((2, _SCH), jnp.int32),
            pltpu.VMEM((_CAP,), jnp.int32),
            pltpu.VMEM((_CAP,), jnp.int32),
            pltpu.VMEM((_CAP,), jnp.int32),
            pltpu.VMEM((_BK, OUT), jnp.float32),
            pltpu.VMEM((_BK, 2 * H), jnp.float32),
            pltpu.VMEM((_BK, 2 * H), jnp.float32),
            pltpu.VMEM((H, _BK), jnp.float32),
            pltpu.VMEM((H, _L), jnp.float32),
            pltpu.SemaphoreType.DMA,
            pltpu.SemaphoreType.DMA,
            pltpu.SemaphoreType.DMA,
        ],
    )
    def sc_edge(hs, tt, srcs, dsts, types, zrows, pacc, dp,
                acc, tbufr, sbufr, dbufr, crs, crd, cdl,
                hbuf, tsb, tdb, wbuf, dacc, sem_t, sem_l, sem_h):
        cid = lax.axis_index("c")
        sid = lax.axis_index("s")
        wid = cid * _NS + sid
        lo = sid * ROWS_PT
        pltpu.sync_copy(zrows, acc)
        lanes = lax.broadcasted_iota(jnp.int32, (_L,), 0)

        zi = jnp.zeros((_L,), jnp.int32)

        def z_body(i, c):
            crs[pl.ds(i * _L, _L)] = zi
            crd[pl.ds(i * _L, _L)] = zi
            cdl[pl.ds(i * _L, _L)] = zi
            return c

        lax.fori_loop(0, _CAP // _L, z_body, 0)
        for h in range(H):
            dacc[h, :] = jnp.zeros((_L,), jnp.float32)

        base_e = cid * EH

        def fire_staging(c, sync=False):
            cb = base_e + c * _SCH
            cp = lax.rem(c, 2)
            copy = pltpu.sync_copy if sync else (
                lambda s, d: pltpu.async_copy(s, d, sem_t))
            copy(types.at[pl.ds(cb, _SCH)], tbufr.at[cp])
            copy(srcs.at[pl.ds(cb, _SCH)], sbufr.at[cp])
            copy(dsts.at[pl.ds(cb, _SCH)], dbufr.at[cp])

        def wait_staging():
            for buf in (tbufr, sbufr, dbufr):
                pltpu.make_async_copy(
                    types.at[pl.ds(0, _SCH)], buf.at[0], sem_t).wait()

        def do_batch(off, valid):
            pltpu.async_copy(hs.at[crs.at[pl.ds(off, _BK)]], hbuf, sem_h)
            pltpu.async_copy(tt.at[crs.at[pl.ds(off, _BK)]], tsb, sem_l)
            pltpu.async_copy(tt.at[crd.at[pl.ds(off, _BK)]], tdb, sem_l)
            pltpu.make_async_copy(tt.at[pl.ds(0, _BK)], tsb, sem_l).wait()
            pltpu.make_async_copy(tt.at[pl.ds(0, _BK)], tdb, sem_l).wait()
            for g in range(_NG):
                rows = g * _L + lanes
                live = (g * _L + lanes) < valid
                for h in range(H):
                    ps = plsc.load_gather(
                        tsb, [rows, jnp.full((_L,), h, jnp.int32)])
                    pd = plsc.load_gather(
                        tdb, [rows, jnp.full((_L,), H + h, jnp.int32)])
                    z = ps + pd
                    w = jnp.exp(jnp.maximum(z, 0.2 * z))
                    w = jnp.where(live, w, 0.0)
                    wbuf[h, pl.ds(g * _L, _L)] = w
                    dacc[h, :] = dacc[h, :] + w
            pltpu.make_async_copy(hs.at[pl.ds(0, _BK)], hbuf, sem_h).wait()
            for g in range(_NG):
                rows = g * _L + lanes
                drow = cdl[pl.ds(off + g * _L, _L)]
                wv = [wbuf[h, pl.ds(g * _L, _L)] for h in range(H)]
                for c0 in range(0, OUT, _L):
                    cols = [plsc.load_gather(
                        hbuf, [rows, jnp.full((_L,), c, jnp.int32)])
                        for c in range(c0, c0 + _L)]
                    for j, c in enumerate(range(c0, c0 + _L)):
                        plsc.addupdate_scatter(
                            acc, [drow, jnp.full((_L,), c, jnp.int32)],
                            cols[j] * wv[c // HD])

        fire_staging(0, sync=True)

        def chunk_body(c_i, cnt):
            cp = lax.rem(c_i, 2)

            @pl.when(c_i > 0)
            def _():
                wait_staging()

            @pl.when(c_i + 1 < NCHK)
            def _():
                fire_staging(c_i + 1)

            def scan_body(g, cn):
                o = g * _L
                t16 = tbufr[cp, pl.ds(o, _L)]
                s16 = sbufr[cp, pl.ds(o, _L)]
                d16 = dbufr[cp, pl.ds(o, _L)]
                m = (d16 >= lo) & (d16 < lo + ROWS_PT)
                mi = m.astype(jnp.int32)
                tN = t16 * N
                idxs = cn + plsc.cumsum(mi) - mi
                plsc.store_scatter(crs, [idxs], tN + s16, mask=m)
                plsc.store_scatter(crd, [idxs], tN + d16, mask=m)
                plsc.store_scatter(cdl, [idxs], d16 - lo, mask=m)
                return cn + plsc.all_reduce_population_count(m)

            cnt = lax.fori_loop(0, _SCH // _L, scan_body, cnt)
            tot = jnp.max(cnt)
            nb = tot // _BK

            def b_body(k, c):
                do_batch(k * _BK, _BK)
                return c

            lax.fori_loop(0, nb, b_body, 0)

            rem = tot - nb * _BK
            for g in range(_NG):
                o = nb * _BK + g * _L
                v1 = crs[pl.ds(o, _L)]
                v2 = crd[pl.ds(o, _L)]
                v3 = cdl[pl.ds(o, _L)]
                crs[pl.ds(g * _L, _L)] = v1
                crd[pl.ds(g * _L, _L)] = v2
                cdl[pl.ds(g * _L, _L)] = v3
            return jnp.full((_L,), rem, jnp.int32)

        cnt = lax.fori_loop(0, NCHK, chunk_body, jnp.zeros((_L,), jnp.int32))
        pend = jnp.max(cnt)

        @pl.when(pend > 0)
        def _():
            do_batch(0, pend)

        pltpu.sync_copy(dacc, dp.at[wid])
        pltpu.sync_copy(acc, pacc.at[wid])

    return sc_edge


def _combine_body(p0_ref, p1_ref, dp_ref, out_ref, *, OUT, H):
    HD = OUT // H
    dsum = jnp.sum(dp_ref[...], axis=(0, 2))
    col = lax.broadcasted_iota(jnp.int32, (1, OUT), 1) // HD
    dvec = jnp.full((1, OUT), 1.0, jnp.float32)
    for h in range(H):
        dvec = jnp.where(col == h, dsum[h], dvec)
    out_ref[0] = (p0_ref[0] + p1_ref[0]) * (1.0 / dvec)


def kernel(x, edge_index, edge_type, W, a_src, a_dst):
    N, IN = x.shape
    R, _, OUT = W.shape
    H, HD = a_src.shape
    E = edge_type.shape[0]

    col = jnp.arange(OUT)
    hsel = (col[:, None] // HD == jnp.arange(H)[None, :]).astype(jnp.float32)
    A = jnp.concatenate(
        [a_src.reshape(-1)[:, None] * hsel,
         a_dst.reshape(-1)[:, None] * hsel], axis=1)

    BN = 2000
    n_blk = N // BN
    hs, tt = pl.pallas_call(
        _proj_body,
        grid=(n_blk, R),
        in_specs=[
            pl.BlockSpec((BN, IN), lambda i, r: (i, 0)),
            pl.BlockSpec((1, IN, OUT), lambda i, r: (r, 0, 0)),
            pl.BlockSpec((IN, 2 * H), lambda i, r: (0, 0)),
        ],
        out_specs=[
            pl.BlockSpec((BN, OUT), lambda i, r: (r * n_blk + i, 0)),
            pl.BlockSpec((BN, 2 * H), lambda i, r: (r * n_blk + i, 0)),
        ],
        out_shape=[
            jax.ShapeDtypeStruct((R * N, OUT), jnp.float32),
            jax.ShapeDtypeStruct((R * N, 2 * H), jnp.float32),
        ],
    )(x, W, A)

    srcs = edge_index[0]
    dsts = edge_index[1]
    zrows = jnp.zeros((N // _NS, OUT), jnp.float32)

    sc_edge = _make_sc_kernel(N, E, OUT, R, H)
    pacc, dp = sc_edge(hs, tt, srcs, dsts, edge_type, zrows)

    RPT = N // _NS
    out = pl.pallas_call(
        functools.partial(_combine_body, OUT=OUT, H=H),
        grid=(_NS,),
        in_specs=[
            pl.BlockSpec((1, RPT, OUT), lambda i: (i, 0, 0)),
            pl.BlockSpec((1, RPT, OUT), lambda i: (i + _NS, 0, 0)),
            pl.BlockSpec((_NW, H, _L), lambda i: (0, 0, 0)),
        ],
        out_specs=pl.BlockSpec((1, RPT, OUT), lambda i: (i, 0, 0)),
        out_shape=jax.ShapeDtypeStruct((_NS, RPT, OUT), jnp.float32),
    )(pacc, pacc, dp)
    return out.reshape(N, OUT)


# speedup vs baseline: 1.0188x; 1.0188x over previous
import functools

import jax
import jax.numpy as jnp
from jax import lax
from jax.experimental import pallas as pl
from jax.experimental.pallas import tpu as pltpu
from jax.experimental.pallas import tpu_sc as plsc

_NC = 2
_NS = 16
_L = 16
_NW = _NC * _NS

_SCH = 2000
_BK = 80
_NG = _BK // _L
_CAP = _SCH + _BK


def _proj_body(x_ref, w_ref, a_ref, h_ref, t_ref):
    h = jnp.dot(x_ref[...], w_ref[0], preferred_element_type=jnp.float32)
    h_ref[...] = h
    t_ref[...] = jnp.dot(h, a_ref[...], preferred_element_type=jnp.float32)


def _make_sc_kernel(N, E, OUT, R, H):
    HD = OUT // H
    EH = E // _NC
    NCHK = EH // _SCH
    ROWS_PT = N // _NS

    mesh = plsc.VectorSubcoreMesh(core_axis_name="c", subcore_axis_name="s")

    @functools.partial(
        pl.kernel,
        out_type=(
            jax.ShapeDtypeStruct((_NW, ROWS_PT, OUT), jnp.float32),
            jax.ShapeDtypeStruct((_NW, H, _L), jnp.float32),
        ),
        mesh=mesh,
        compiler_params=pltpu.CompilerParams(
            needs_layout_passes=False, use_tc_tiling_on_sc=False),
        scratch_types=[
            pltpu.VMEM((ROWS_PT, OUT), jnp.float32),
            pltpu.VMEM((2, _SCH), jnp.int32),
            pltpu.VMEM((2, _SCH), jnp.int32),
            pltpu.VMEM((2, _SCH), jnp.int32),
            pltpu.VMEM((_CAP,), jnp.int32),
            pltpu.VMEM((_CAP,), jnp.int32),
            pltpu.VMEM((_CAP,), jnp.int32),
            pltpu.VMEM((2, _BK, OUT), jnp.float32),
            pltpu.VMEM((2, _BK, 2 * H), jnp.float32),
            pltpu.VMEM((2, _BK, 2 * H), jnp.float32),
            pltpu.VMEM((H, _BK), jnp.float32),
            pltpu.VMEM((H, _L), jnp.float32),
            pltpu.SemaphoreType.DMA,
            pltpu.SemaphoreType.DMA,
            pltpu.SemaphoreType.DMA,
            pltpu.SemaphoreType.DMA,
            pltpu.SemaphoreType.DMA,
        ],
    )
    def sc_edge(hs, tt, srcs, dsts, types, zrows, pacc, dp,
                acc, tbufr, sbufr, dbufr, crs, crd, cdl,
                hbufr, tsbr, tdbr, wbuf, dacc,
                sem_t, sem_l0, sem_l1, sem_h0, sem_h1):
        cid = lax.axis_index("c")
        sid = lax.axis_index("s")
        wid = cid * _NS + sid
        lo = sid * ROWS_PT
        pltpu.sync_copy(zrows, acc)
        lanes = lax.broadcasted_iota(jnp.int32, (_L,), 0)

        zi = jnp.zeros((_L,), jnp.int32)

        def z_body(i, c):
            crs[pl.ds(i * _L, _L)] = zi
            crd[pl.ds(i * _L, _L)] = zi
            cdl[pl.ds(i * _L, _L)] = zi
            return c

        lax.fori_loop(0, _CAP // _L, z_body, 0)
        for h in range(H):
            dacc[h, :] = jnp.zeros((_L,), jnp.float32)

        base_e = cid * EH

        def fire_staging(c, sync=False):
            cb = base_e + c * _SCH
            cp = lax.rem(c, 2)
            copy = pltpu.sync_copy if sync else (
                lambda s, d: pltpu.async_copy(s, d, sem_t))
            copy(types.at[pl.ds(cb, _SCH)], tbufr.at[cp])
            copy(srcs.at[pl.ds(cb, _SCH)], sbufr.at[cp])
            copy(dsts.at[pl.ds(cb, _SCH)], dbufr.at[cp])

        def wait_staging():
            for buf in (tbufr, sbufr, dbufr):
                pltpu.make_async_copy(
                    types.at[pl.ds(0, _SCH)], buf.at[0], sem_t).wait()

        def fire_batch(off, q):
            @pl.when(q == 0)
            def _():
                pltpu.async_copy(hs.at[crs.at[pl.ds(off, _BK)]],
                                 hbufr.at[0], sem_h0)
                pltpu.async_copy(tt.at[crs.at[pl.ds(off, _BK)]],
                                 tsbr.at[0], sem_l0)
                pltpu.async_copy(tt.at[crd.at[pl.ds(off, _BK)]],
                                 tdbr.at[0], sem_l0)

            @pl.when(q == 1)
            def _():
                pltpu.async_copy(hs.at[crs.at[pl.ds(off, _BK)]],
                                 hbufr.at[1], sem_h1)
                pltpu.async_copy(tt.at[crs.at[pl.ds(off, _BK)]],
                                 tsbr.at[1], sem_l1)
                pltpu.async_copy(tt.at[crd.at[pl.ds(off, _BK)]],
                                 tdbr.at[1], sem_l1)

        def proc_batch(off, valid, q):
            @pl.when(q == 0)
            def _():
                pltpu.make_async_copy(
                    tt.at[pl.ds(0, _BK)], tsbr.at[0], sem_l0).wait()
                pltpu.make_async_copy(
                    tt.at[pl.ds(0, _BK)], tdbr.at[0], sem_l0).wait()

            @pl.when(q == 1)
            def _():
                pltpu.make_async_copy(
                    tt.at[pl.ds(0, _BK)], tsbr.at[1], sem_l1).wait()
                pltpu.make_async_copy(
                    tt.at[pl.ds(0, _BK)], tdbr.at[1], sem_l1).wait()

            tsb = tsbr.at[q]
            tdb = tdbr.at[q]
            for g in range(_NG):
                rows = g * _L + lanes
                live = (g * _L + lanes) < valid
                for h in range(H):
                    ps = plsc.load_gather(
                        tsb, [rows, jnp.full((_L,), h, jnp.int32)])
                    pd = plsc.load_gather(
                        tdb, [rows, jnp.full((_L,), H + h, jnp.int32)])
                    z = ps + pd
                    w = jnp.exp(jnp.maximum(z, 0.2 * z))
                    w = jnp.where(live, w, 0.0)
                    wbuf[h, pl.ds(g * _L, _L)] = w
                    dacc[h, :] = dacc[h, :] + w

            @pl.when(q == 0)
            def _():
                pltpu.make_async_copy(
                    hs.at[pl.ds(0, _BK)], hbufr.at[0], sem_h0).wait()

            @pl.when(q == 1)
            def _():
                pltpu.make_async_copy(
                    hs.at[pl.ds(0, _BK)], hbufr.at[1], sem_h1).wait()

            hbuf = hbufr.at[q]
            for g in range(_NG):
                rows = g * _L + lanes
                drow = cdl[pl.ds(off + g * _L, _L)]
                wv = [wbuf[h, pl.ds(g * _L, _L)] for h in range(H)]
                for c0 in range(0, OUT, _L):
                    cols = [plsc.load_gather(
                        hbuf, [rows, jnp.full((_L,), c, jnp.int32)])
                        for c in range(c0, c0 + _L)]
                    for j, c in enumerate(range(c0, c0 + _L)):
                        plsc.addupdate_scatter(
                            acc, [drow, jnp.full((_L,), c, jnp.int32)],
                            cols[j] * wv[c // HD])

        fire_staging(0, sync=True)

        def chunk_body(c_i, cnt):
            cp = lax.rem(c_i, 2)

            @pl.when(c_i > 0)
            def _():
                wait_staging()

            @pl.when(c_i + 1 < NCHK)
            def _():
                fire_staging(c_i + 1)

            def scan_body(g5, cn):
                for gg in range(5):
                    o = (g5 * 5 + gg) * _L
                    t16 = tbufr[cp, pl.ds(o, _L)]
                    s16 = sbufr[cp, pl.ds(o, _L)]
                    d16 = dbufr[cp, pl.ds(o, _L)]
                    m = (d16 >= lo) & (d16 < lo + ROWS_PT)
                    mi = m.astype(jnp.int32)
                    tN = t16 * N
                    idxs = cn + plsc.cumsum(mi) - mi
                    plsc.store_scatter(crs, [idxs], tN + s16, mask=m)
                    plsc.store_scatter(crd, [idxs], tN + d16, mask=m)
                    plsc.store_scatter(cdl, [idxs], d16 - lo, mask=m)
                    cn = cn + plsc.all_reduce_population_count(m)
                return cn

            cnt = lax.fori_loop(0, _SCH // _L // 5, scan_body, cnt)
            tot = jnp.max(cnt)
            nb = tot // _BK

            @pl.when(nb > 0)
            def _():
                fire_batch(0, 0)

            def b_body(k, c):
                q = lax.rem(k, 2)

                @pl.when(k + 1 < nb)
                def _():
                    fire_batch((k + 1) * _BK, lax.rem(k + 1, 2))

                proc_batch(k * _BK, _BK, q)
                return c

            lax.fori_loop(0, nb, b_body, 0)

            rem = tot - nb * _BK
            for g in range(_NG):
                o = nb * _BK + g * _L
                v1 = crs[pl.ds(o, _L)]
                v2 = crd[pl.ds(o, _L)]
                v3 = cdl[pl.ds(o, _L)]
                crs[pl.ds(g * _L, _L)] = v1
                crd[pl.ds(g * _L, _L)] = v2
                cdl[pl.ds(g * _L, _L)] = v3
            return jnp.full((_L,), rem, jnp.int32)

        cnt = lax.fori_loop(0, NCHK, chunk_body, jnp.zeros((_L,), jnp.int32))
        pend = jnp.max(cnt)

        @pl.when(pend > 0)
        def _():
            fire_batch(0, 0)
            proc_batch(0, pend, 0)

        pltpu.sync_copy(dacc, dp.at[wid])
        pltpu.sync_copy(acc, pacc.at[wid])

    return sc_edge


def _combine_body(p0_ref, p1_ref, dp_ref, out_ref, *, OUT, H):
    HD = OUT // H
    dsum = jnp.sum(dp_ref[...], axis=(0, 2))
    col = lax.broadcasted_iota(jnp.int32, (1, OUT), 1) // HD
    dvec = jnp.full((1, OUT), 1.0, jnp.float32)
    for h in range(H):
        dvec = jnp.where(col == h, dsum[h], dvec)
    out_ref[0] = (p0_ref[0] + p1_ref[0]) * (1.0 / dvec)


def kernel(x, edge_index, edge_type, W, a_src, a_dst):
    N, IN = x.shape
    R, _, OUT = W.shape
    H, HD = a_src.shape
    E = edge_type.shape[0]

    col = jnp.arange(OUT)
    hsel = (col[:, None] // HD == jnp.arange(H)[None, :]).astype(jnp.float32)
    A = jnp.concatenate(
        [a_src.reshape(-1)[:, None] * hsel,
         a_dst.reshape(-1)[:, None] * hsel], axis=1)

    BN = 2000
    n_blk = N // BN
    hs, tt = pl.pallas_call(
        _proj_body,
        grid=(n_blk, R),
        in_specs=[
            pl.BlockSpec((BN, IN), lambda i, r: (i, 0)),
            pl.BlockSpec((1, IN, OUT), lambda i, r: (r, 0, 0)),
            pl.BlockSpec((IN, 2 * H), lambda i, r: (0, 0)),
        ],
        out_specs=[
            pl.BlockSpec((BN, OUT), lambda i, r: (r * n_blk + i, 0)),
            pl.BlockSpec((BN, 2 * H), lambda i, r: (r * n_blk + i, 0)),
        ],
        out_shape=[
            jax.ShapeDtypeStruct((R * N, OUT), jnp.float32),
            jax.ShapeDtypeStruct((R * N, 2 * H), jnp.float32),
        ],
    )(x, W, A)

    srcs = edge_index[0]
    dsts = edge_index[1]
    zrows = jnp.zeros((N // _NS, OUT), jnp.float32)

    sc_edge = _make_sc_kernel(N, E, OUT, R, H)
    pacc, dp = sc_edge(hs, tt, srcs, dsts, edge_type, zrows)

    RPT = N // _NS
    out = pl.pallas_call(
        functools.partial(_combine_body, OUT=OUT, H=H),
        grid=(_NS,),
        in_specs=[
            pl.BlockSpec((1, RPT, OUT), lambda i: (i, 0, 0)),
            pl.BlockSpec((1, RPT, OUT), lambda i: (i + _NS, 0, 0)),
            pl.BlockSpec((_NW, H, _L), lambda i: (0, 0, 0)),
        ],
        out_specs=pl.BlockSpec((1, RPT, OUT), lambda i: (i, 0, 0)),
        out_shape=jax.ShapeDtypeStruct((_NS, RPT, OUT), jnp.float32),
    )(pacc, pacc, dp)
    return out.reshape(N, OUT)


# speedup vs baseline: 4.4121x; 4.3306x over previous
import functools

import jax
import jax.numpy as jnp
from jax import lax
from jax.experimental import pallas as pl
from jax.experimental.pallas import tpu as pltpu
from jax.experimental.pallas import tpu_sc as plsc

_NC = 2
_NS = 16
_L = 16
_NW = _NC * _NS

_CH = 400
_BK = 80
_NB = _CH // _BK
_NG = _BK // _L


def _proj_body(x_ref, w_ref, a_ref, h_ref, t_ref):
    h = jnp.dot(x_ref[...], w_ref[0], preferred_element_type=jnp.float32)
    h_ref[...] = h
    t_ref[...] = jnp.dot(h, a_ref[...], preferred_element_type=jnp.float32)


def _make_sc_kernel(N, E, OUT, R, H):
    HD = OUT // H
    EPW = E // _NW
    NCH = EPW // _CH
    ROWS_PT = N // _NS

    mesh = plsc.VectorSubcoreMesh(core_axis_name="c", subcore_axis_name="s")

    @functools.partial(
        pl.kernel,
        out_type=(
            jax.ShapeDtypeStruct((_NW, ROWS_PT, OUT), jnp.float32),
            jax.ShapeDtypeStruct((_NW, H, _L), jnp.float32),
        ),
        mesh=mesh,
        compiler_params=pltpu.CompilerParams(
            needs_layout_passes=False, use_tc_tiling_on_sc=False),
        scratch_types=[
            pltpu.VMEM_SHARED((N, OUT), jnp.float32),
            pltpu.VMEM((_CH,), jnp.int32),
            pltpu.VMEM((_CH,), jnp.int32),
            pltpu.VMEM((_CH,), jnp.int32),
            pltpu.VMEM((_NB, _BK), jnp.int32),
            pltpu.VMEM((_BK, OUT), jnp.float32),
            pltpu.VMEM((_BK, OUT), jnp.float32),
            pltpu.VMEM((_CH, 2 * H), jnp.float32),
            pltpu.VMEM((_CH, 2 * H), jnp.float32),
            pltpu.VMEM((H, _CH), jnp.float32),
            pltpu.VMEM((H, _L), jnp.float32),
            pltpu.SemaphoreType.DMA,
            pltpu.SemaphoreType.DMA,
            pltpu.SemaphoreType.DMA,
            pltpu.SemaphoreType.DMA,
            pltpu.SemaphoreType.DMA,
        ],
    )
    def sc_edge(hs, tt, srcs, dsts, types, zrows, pacc, dp,
                acc_sh, tbuf, sbuf, dbuf, dbuf2,
                hrows0, hrows1, tsb, tdb, wbuf,
                dacc, sem_s, sem_b0, sem_b1, sem_c0, sem_c1):
        cid = lax.axis_index("c")
        sid = lax.axis_index("s")
        wid = cid * _NS + sid
        pltpu.sync_copy(zrows, acc_sh.at[pl.ds(sid * ROWS_PT, ROWS_PT)])
        plsc.subcore_barrier()

        base_e = wid * EPW
        lanes = lax.broadcasted_iota(jnp.int32, (_L,), 0)
        zero16 = jnp.zeros((_L,), jnp.float32)

        def fire_hrow(b, buf, sem):
            pltpu.async_copy(hs.at[sbuf.at[pl.ds(b * _BK, _BK)]], buf, sem)

        def wait_hrow(buf, sem):
            pltpu.make_async_copy(hs.at[pl.ds(0, _BK)], buf, sem).wait()

        def wait_scat(buf, sem):
            pltpu.make_async_copy(buf, acc_sh.at[pl.ds(0, _BK)], sem).wait()

        def mul_scat(b, buf, sem):
            off = b * _BK

            def grp_body(g, carry):
                wv = [wbuf[h, pl.ds(off + g * _L, _L)] for h in range(H)]
                for j in range(_L):
                    r = g * _L + j
                    for h in range(H):
                        w = wv[h][j]
                        for c0 in range(h * HD, (h + 1) * HD, _L):
                            buf[r, pl.ds(c0, _L)] = (
                                buf[r, pl.ds(c0, _L)] * w)
                return carry

            lax.fori_loop(0, _NG, grp_body, 0)
            pltpu.async_copy(buf, acc_sh.at[dbuf2.at[b]], sem, add=True)

        def chunk_body(c_i, dcarry):
            cb = base_e + c_i * _CH
            pltpu.sync_copy(types.at[pl.ds(cb, _CH)], tbuf)
            pltpu.sync_copy(srcs.at[pl.ds(cb, _CH)], sbuf)
            pltpu.sync_copy(dsts.at[pl.ds(cb, _CH)], dbuf)

            def idx_body(b, carry):
                for g in range(_NG):
                    o = b * _BK + g * _L
                    t16 = tbuf[pl.ds(o, _L)]
                    s16 = sbuf[pl.ds(o, _L)]
                    d16 = dbuf[pl.ds(o, _L)]
                    tN = t16 * N
                    dbuf2[b, pl.ds(g * _L, _L)] = d16
                    sbuf[pl.ds(o, _L)] = tN + s16
                    dbuf[pl.ds(o, _L)] = tN + d16
                off = b * _BK
                pltpu.async_copy(tt.at[sbuf.at[pl.ds(off, _BK)]],
                                 tsb.at[pl.ds(off, _BK)], sem_s)
                pltpu.async_copy(tt.at[dbuf.at[pl.ds(off, _BK)]],
                                 tdb.at[pl.ds(off, _BK)], sem_s)
                return carry

            lax.fori_loop(0, _NB, idx_body, 0)
            fire_hrow(0, hrows0, sem_b0)
            pltpu.make_async_copy(tt.at[pl.ds(0, _CH)], tsb, sem_s).wait()
            pltpu.make_async_copy(tt.at[pl.ds(0, _CH)], tdb, sem_s).wait()

            def w_body(g, dc):
                rows = g * _L + lanes
                ws = []
                for h in range(H):
                    ps = plsc.load_gather(
                        tsb, [rows, jnp.full((_L,), h, jnp.int32)])
                    pd = plsc.load_gather(
                        tdb, [rows, jnp.full((_L,), H + h, jnp.int32)])
                    z = ps + pd
                    w = jnp.exp(jnp.maximum(z, 0.2 * z))
                    wbuf[h, pl.ds(g * _L, _L)] = w
                    ws.append(w)
                return tuple(dc[h] + ws[h] for h in range(H))

            dcarry = lax.fori_loop(0, _CH // _L, w_body, dcarry)

            def pipe_body(i, dc):
                b = 2 * i

                @pl.when(i > 0)
                def _():
                    wait_scat(hrows1, sem_c1)

                @pl.when(b + 1 < _NB)
                def _():
                    fire_hrow(b + 1, hrows1, sem_b1)

                wait_hrow(hrows0, sem_b0)
                mul_scat(b, hrows0, sem_c0)

                @pl.when(b + 1 < _NB)
                def _():
                    wait_scat(hrows0, sem_c0)
                    fire_hrow(b + 2, hrows0, sem_b0)
                    wait_hrow(hrows1, sem_b1)
                    mul_scat(b + 1, hrows1, sem_c1)

                return dc

            dcarry = lax.fori_loop(0, (_NB + 1) // 2, pipe_body, dcarry)
            wait_scat(hrows0, sem_c0)
            return dcarry

        dfin = lax.fori_loop(0, NCH, chunk_body, (zero16,) * H)
        for h in range(H):
            dacc[h, :] = dfin[h]
        pltpu.sync_copy(dacc, dp.at[wid])
        plsc.subcore_barrier()
        pltpu.sync_copy(acc_sh.at[pl.ds(sid * ROWS_PT, ROWS_PT)],
                        pacc.at[wid])

    return sc_edge


def _combine_body(p0_ref, p1_ref, dp_ref, out_ref, *, OUT, H):
    HD = OUT // H
    dsum = jnp.sum(dp_ref[...], axis=(0, 2))
    col = lax.broadcasted_iota(jnp.int32, (1, OUT), 1) // HD
    dvec = jnp.full((1, OUT), 1.0, jnp.float32)
    for h in range(H):
        dvec = jnp.where(col == h, dsum[h], dvec)
    out_ref[0] = (p0_ref[0] + p1_ref[0]) * (1.0 / dvec)


def kernel(x, edge_index, edge_type, W, a_src, a_dst):
    N, IN = x.shape
    R, _, OUT = W.shape
    H, HD = a_src.shape
    E = edge_type.shape[0]

    col = jnp.arange(OUT)
    hsel = (col[:, None] // HD == jnp.arange(H)[None, :]).astype(jnp.float32)
    A = jnp.concatenate(
        [a_src.reshape(-1)[:, None] * hsel,
         a_dst.reshape(-1)[:, None] * hsel], axis=1)

    BN = 2000
    n_blk = N // BN
    hs, tt = pl.pallas_call(
        _proj_body,
        grid=(n_blk, R),
        in_specs=[
            pl.BlockSpec((BN, IN), lambda i, r: (i, 0)),
            pl.BlockSpec((1, IN, OUT), lambda i, r: (r, 0, 0)),
            pl.BlockSpec((IN, 2 * H), lambda i, r: (0, 0)),
        ],
        out_specs=[
            pl.BlockSpec((BN, OUT), lambda i, r: (r * n_blk + i, 0)),
            pl.BlockSpec((BN, 2 * H), lambda i, r: (r * n_blk + i, 0)),
        ],
        out_shape=[
            jax.ShapeDtypeStruct((R * N, OUT), jnp.float32),
            jax.ShapeDtypeStruct((R * N, 2 * H), jnp.float32),
        ],
    )(x, W, A)

    srcs = edge_index[0]
    dsts = edge_index[1]
    zrows = jnp.zeros((N // _NS, OUT), jnp.float32)

    sc_edge = _make_sc_kernel(N, E, OUT, R, H)
    pacc, dp = sc_edge(hs, tt, srcs, dsts, edge_type, zrows)

    RPT = N // _NS
    out = pl.pallas_call(
        functools.partial(_combine_body, OUT=OUT, H=H),
        grid=(_NS,),
        in_specs=[
            pl.BlockSpec((1, RPT, OUT), lambda i: (i, 0, 0)),
            pl.BlockSpec((1, RPT, OUT), lambda i: (i + _NS, 0, 0)),
            pl.BlockSpec((_NW, H, _L), lambda i: (0, 0, 0)),
        ],
        out_specs=pl.BlockSpec((1, RPT, OUT), lambda i: (i, 0, 0)),
        out_shape=jax.ShapeDtypeStruct((_NS, RPT, OUT), jnp.float32),
    )(pacc, pacc, dp)
    return out.reshape(N, OUT)
